# Initial kernel scaffold; baseline (speedup 1.0000x reference)
import jax
import jax.numpy as jnp
from jax.experimental import pallas as pl


def kernel(box_cls, box_regression, iou_pred, anchors):
    raise NotImplementedError("write your pallas kernel here")


# speedup vs baseline: 1.0038x; 1.0038x over previous
import jax
import jax.numpy as jnp
from jax.experimental import pallas as pl
from jax.experimental.pallas import tpu as pltpu

N, C, A, H, W = 2, 80, 1, 128, 128
HWA = H * W * A
M = HWA * C
STRIDE = 8
IMG = 1024
PRE_NMS_THRESH = 0.05
PRE_NMS_TOP_N = 1000
MIN_SIZE = 0.0
WX, WY, WW, WH = 10.0, 10.0, 5.0, 5.0
import numpy as _np
BBOX_XFORM_CLIP = float(_np.log(1000.0 / 16.0))
TOPK = min(PRE_NMS_TOP_N * 10, M)

BLK = 1024


def _score_body(cls_ref, iou_ref, out_ref):
    bc = jax.nn.sigmoid(cls_ref[0])
    iou = jax.nn.sigmoid(iou_ref[0, 0, 0])
    s = jnp.sqrt(bc * iou[:, None])
    out_ref[0] = jnp.where(bc > PRE_NMS_THRESH, s, -1.0)


def _masked_scores(box_cls, iou_pred):
    bc_t = jnp.transpose(box_cls.reshape(N, C, HWA), (0, 2, 1))
    iou_t = iou_pred.reshape(N, HWA // BLK, 1, BLK)
    grid = (N, HWA // BLK)
    out = pl.pallas_call(
        _score_body,
        grid=grid,
        in_specs=[
            pl.BlockSpec((1, BLK, C), lambda n, b: (n, b, 0)),
            pl.BlockSpec((1, 1, 1, BLK), lambda n, b: (n, b, 0, 0)),
        ],
        out_specs=pl.BlockSpec((1, BLK, C), lambda n, b: (n, b, 0)),
        out_shape=jax.ShapeDtypeStruct((N, HWA, C), jnp.float32),
    )(bc_t, iou_t)
    return out.reshape(N, M)


def _decode(rel, anchors):
    TO_REMOVE = 1.0
    widths = anchors[..., 2] - anchors[..., 0] + TO_REMOVE
    heights = anchors[..., 3] - anchors[..., 1] + TO_REMOVE
    ctr_x = anchors[..., 0] + 0.5 * widths
    ctr_y = anchors[..., 1] + 0.5 * heights
    dx = rel[..., 0] / WX
    dy = rel[..., 1] / WY
    dw = jnp.minimum(rel[..., 2] / WW, BBOX_XFORM_CLIP)
    dh = jnp.minimum(rel[..., 3] / WH, BBOX_XFORM_CLIP)
    pred_ctr_x = dx * widths + ctr_x
    pred_ctr_y = dy * heights + ctr_y
    pred_w = jnp.exp(dw) * widths
    pred_h = jnp.exp(dh) * heights
    x1 = pred_ctr_x - 0.5 * (pred_w - 1.0)
    y1 = pred_ctr_y - 0.5 * (pred_h - 1.0)
    x2 = pred_ctr_x + 0.5 * (pred_w - 1.0)
    y2 = pred_ctr_y + 0.5 * (pred_h - 1.0)
    return jnp.stack([x1, y1, x2, y2], axis=-1)


def kernel(box_cls, box_regression, iou_pred, anchors):
    masked = _masked_scores(box_cls, iou_pred)
    top_v, top_i = jax.lax.top_k(masked, TOPK)
    loc = top_i // C
    cls = top_i % C + 1
    br = jnp.transpose(box_regression.reshape(N, 4, HWA), (0, 2, 1))
    per_reg = jnp.take_along_axis(br, loc[..., None], axis=1)
    per_anc = jnp.take_along_axis(anchors, loc[..., None], axis=1)
    det = _decode(per_reg, per_anc)
    det = jnp.clip(det, 0.0, IMG - 1.0)
    ws = det[..., 2] - det[..., 0] + 1.0
    hs = det[..., 3] - det[..., 1] + 1.0
    valid = (top_v > 0.0) & (ws >= MIN_SIZE) & (hs >= MIN_SIZE)
    scores_out = jnp.where(valid, top_v, 0.0)
    labels = jnp.where(valid, cls, 0)
    return det, scores_out, labels
